# baseline (speedup 1.0000x reference)
import functools
import math

import jax
import jax.numpy as jnp
from jax.experimental import pallas as pl
from jax.experimental.pallas import tpu as pltpu
from jax.experimental.pallas import tpu_sc as plsc

SPAT_DIM = 48
TIME_DIM = 48
MAX_N_TIMES = 30000
NUM_CHANNELS = 64

_CPT = 16


def _time_encoding(n_times: int) -> jnp.ndarray:
    position = jnp.arange(n_times, dtype=jnp.float32)[:, None]
    div_term = jnp.exp(
        jnp.arange(0, TIME_DIM, 2, dtype=jnp.float32)
        * (-math.log(MAX_N_TIMES) / TIME_DIM)
    )
    s = jnp.sin(position * div_term)
    c = jnp.cos(position * div_term)
    return jnp.stack([s, c], axis=-1).reshape(n_times, TIME_DIM)


def _sc_gather(table, idx_flat):
    info = plsc.get_sparse_core_info()
    num_workers = info.num_cores * info.num_subcores
    n_idx = idx_flat.shape[0]
    per_w = n_idx // num_workers
    mesh = plsc.VectorSubcoreMesh(core_axis_name="c", subcore_axis_name="s")

    @functools.partial(
        pl.kernel,
        mesh=mesh,
        out_type=jax.ShapeDtypeStruct((n_idx, 128), jnp.float32),
        scratch_types=[
            pltpu.VMEM((per_w,), jnp.int32),
            pltpu.VMEM((per_w, 128), jnp.float32),
            pltpu.SemaphoreType.DMA,
        ],
    )
    def gather_kernel(table_hbm, idx_hbm, out_hbm, idx_v, rows_v, sem):
        wid = jax.lax.axis_index("s") * info.num_cores + jax.lax.axis_index("c")
        base = wid * per_w
        pltpu.sync_copy(idx_hbm.at[pl.ds(base, per_w)], idx_v)
        pltpu.async_copy(table_hbm.at[idx_v], rows_v, sem).wait()
        pltpu.sync_copy(rows_v, out_hbm.at[pl.ds(base, per_w)])

    return gather_kernel(table, idx_flat)


def _encode_kernel(spat_ref, tt_ref, out_ref):
    n_times = tt_ref.shape[1] // _CPT
    out_ref[0, pl.ds(SPAT_DIM, TIME_DIM), :] = tt_ref[:, :]
    cols = jnp.transpose(spat_ref[:, 0:SPAT_DIM])
    lane_iota = jax.lax.broadcasted_iota(jnp.int32, (SPAT_DIM, _CPT), 1)
    for k in range(_CPT):
        col = jnp.sum(
            jnp.where(lane_iota == k, cols, 0.0), axis=1, keepdims=True
        )
        out_ref[0, pl.ds(0, SPAT_DIM), pl.ds(k * n_times, n_times)] = (
            jnp.broadcast_to(col, (SPAT_DIM, n_times))
        )


def kernel(local_features, ch_idxs, W_spat):
    batch_size, n_chans_times, emb_dim = local_features.shape
    _, n_chans = ch_idxs.shape
    n_times = n_chans_times // n_chans
    t_enc = _time_encoding(n_times)
    tt = jnp.tile(t_enc.T, (1, _CPT))

    wpad = jnp.pad(W_spat, ((0, 0), (0, 128 - SPAT_DIM)))
    spat = _sc_gather(wpad, ch_idxs.reshape(-1))

    tiles_per_batch = n_chans // _CPT
    cols = _CPT * n_times
    out_t = pl.pallas_call(
        _encode_kernel,
        grid=(batch_size, tiles_per_batch),
        in_specs=[
            pl.BlockSpec(
                (_CPT, 128),
                lambda b, j: (b * (NUM_CHANNELS // _CPT) + j, 0),
            ),
            pl.BlockSpec((TIME_DIM, cols), lambda b, j: (0, 0)),
        ],
        out_specs=pl.BlockSpec((1, emb_dim, cols), lambda b, j: (b, 0, j)),
        out_shape=jax.ShapeDtypeStruct(
            (batch_size, emb_dim, n_chans_times), jnp.float32
        ),
    )(spat, tt)
    return out_t.transpose(0, 2, 1)


# speedup vs baseline: 1.3613x; 1.3613x over previous
import math

import jax
import jax.numpy as jnp
from jax.experimental import pallas as pl
from jax.experimental.pallas import tpu as pltpu

SPAT_DIM = 48
TIME_DIM = 48
MAX_N_TIMES = 30000
NUM_CHANNELS = 64

_CPT = 16


def _time_encoding(n_times: int) -> jnp.ndarray:
    position = jnp.arange(n_times, dtype=jnp.float32)[:, None]
    div_term = jnp.exp(
        jnp.arange(0, TIME_DIM, 2, dtype=jnp.float32)
        * (-math.log(MAX_N_TIMES) / TIME_DIM)
    )
    s = jnp.sin(position * div_term)
    c = jnp.cos(position * div_term)
    return jnp.stack([s, c], axis=-1).reshape(n_times, TIME_DIM)


def _encode_kernel(idx_ref, wt_ref, tt_ref, out_ref):
    b = pl.program_id(0)
    j = pl.program_id(1)
    n_times = tt_ref.shape[1] // _CPT
    out_ref[0, pl.ds(SPAT_DIM, TIME_DIM), :] = tt_ref[:, :]
    chan_iota = jax.lax.broadcasted_iota(
        jnp.int32, (SPAT_DIM, NUM_CHANNELS), 1
    )
    wt = wt_ref[:, :]
    for k in range(_CPT):
        cidx = idx_ref[b, j * _CPT + k]
        col = jnp.sum(
            jnp.where(chan_iota == cidx, wt, 0.0), axis=1, keepdims=True
        )
        out_ref[0, pl.ds(0, SPAT_DIM), pl.ds(k * n_times, n_times)] = (
            jnp.broadcast_to(col, (SPAT_DIM, n_times))
        )


def kernel(local_features, ch_idxs, W_spat):
    batch_size, n_chans_times, emb_dim = local_features.shape
    _, n_chans = ch_idxs.shape
    n_times = n_chans_times // n_chans
    t_enc = _time_encoding(n_times)
    wt = W_spat.T
    tt = jnp.tile(t_enc.T, (1, _CPT))

    tiles_per_batch = n_chans // _CPT
    cols = _CPT * n_times
    grid_spec = pltpu.PrefetchScalarGridSpec(
        num_scalar_prefetch=1,
        grid=(batch_size, tiles_per_batch),
        in_specs=[
            pl.BlockSpec((SPAT_DIM, NUM_CHANNELS), lambda b, j, idx: (0, 0)),
            pl.BlockSpec((TIME_DIM, cols), lambda b, j, idx: (0, 0)),
        ],
        out_specs=pl.BlockSpec((1, emb_dim, cols), lambda b, j, idx: (b, 0, j)),
    )
    out_t = pl.pallas_call(
        _encode_kernel,
        grid_spec=grid_spec,
        out_shape=jax.ShapeDtypeStruct(
            (batch_size, emb_dim, n_chans_times), jnp.float32
        ),
    )(ch_idxs, wt, tt)
    return out_t.transpose(0, 2, 1)
